# baseline (speedup 1.0000x reference)
import functools

import jax
import jax.numpy as jnp
from jax import lax
from jax.experimental import pallas as pl
from jax.experimental.pallas import tpu as pltpu
from jax.experimental.pallas import tpu_sc as plsc

VOCAB = 1000000
EMBED_DIM = 64
BATCH = 16384

NUM_CORES = 2
NUM_SUBCORES = 16
NUM_WORKERS = NUM_CORES * NUM_SUBCORES
B_PER_W = BATCH // NUM_WORKERS
CHUNK = 128
NCHUNK = B_PER_W // CHUNK
LANES = 16
BLOCKS = B_PER_W // LANES


def _body(tgt_idx_hbm, ctx_idx_hbm, tgt_tab_hbm, ctx_tab_hbm, out_hbm,
          tidx_v, cidx_v, trows_v, crows_v, out_v, sem_t, sem_c):
    wid = lax.axis_index("s") * NUM_CORES + lax.axis_index("c")
    base = wid * B_PER_W

    for j in range(NCHUNK):
        pltpu.sync_copy(tgt_idx_hbm.at[pl.ds(base + j * CHUNK, CHUNK)],
                        tidx_v.at[j])
        pltpu.sync_copy(ctx_idx_hbm.at[pl.ds(base + j * CHUNK, CHUNK)],
                        cidx_v.at[j])

    copies = []
    for j in range(NCHUNK):
        copies.append(pltpu.async_copy(
            tgt_tab_hbm.at[tidx_v.at[j]],
            trows_v.at[pl.ds(j * CHUNK, CHUNK)], sem_t))
        copies.append(pltpu.async_copy(
            ctx_tab_hbm.at[cidx_v.at[j]],
            crows_v.at[pl.ds(j * CHUNK, CHUNK)], sem_c))
    for c in copies:
        c.wait()

    def block_step(blk, _):
        rows16 = lax.iota(jnp.int32, LANES) + blk * LANES
        acc0 = jnp.zeros((LANES,), jnp.float32)
        acc1 = jnp.zeros((LANES,), jnp.float32)
        for d in range(0, EMBED_DIM, 2):
            d0 = jnp.full((LANES,), d, jnp.int32)
            d1 = jnp.full((LANES,), d + 1, jnp.int32)
            t0 = plsc.load_gather(trows_v, [rows16, d0])
            c0 = plsc.load_gather(crows_v, [rows16, d0])
            t1 = plsc.load_gather(trows_v, [rows16, d1])
            c1 = plsc.load_gather(crows_v, [rows16, d1])
            acc0 = acc0 + t0 * c0
            acc1 = acc1 + t1 * c1
        out_v[pl.ds(blk * LANES, LANES)] = acc0 + acc1
        return 0

    lax.fori_loop(0, BLOCKS, block_step, 0)

    pltpu.sync_copy(out_v, out_hbm.at[pl.ds(base, B_PER_W)])


@functools.partial(jax.jit, static_argnums=())
def _run(target, context, target_table, context_table):
    mesh = plsc.VectorSubcoreMesh(core_axis_name="c", subcore_axis_name="s")
    kfn = pl.kernel(
        _body,
        mesh=mesh,
        compiler_params=pltpu.CompilerParams(
            needs_layout_passes=False, use_tc_tiling_on_sc=False),
        out_type=jax.ShapeDtypeStruct((BATCH,), jnp.float32),
        scratch_types=[
            pltpu.VMEM((NCHUNK, CHUNK), jnp.int32),
            pltpu.VMEM((NCHUNK, CHUNK), jnp.int32),
            pltpu.VMEM((B_PER_W, EMBED_DIM), jnp.float32),
            pltpu.VMEM((B_PER_W, EMBED_DIM), jnp.float32),
            pltpu.VMEM((B_PER_W,), jnp.float32),
            pltpu.SemaphoreType.DMA,
            pltpu.SemaphoreType.DMA,
        ],
    )
    return kfn(target, context, target_table, context_table)


def kernel(target, context, target_table, context_table):
    return _run(target.astype(jnp.int32), context.astype(jnp.int32),
                target_table, context_table)


# speedup vs baseline: 1.0043x; 1.0043x over previous
import functools

import jax
import jax.numpy as jnp
from jax import lax
from jax.experimental import pallas as pl
from jax.experimental.pallas import tpu as pltpu
from jax.experimental.pallas import tpu_sc as plsc

VOCAB = 1000000
EMBED_DIM = 64
BATCH = 16384

NUM_CORES = 2
NUM_SUBCORES = 16
NUM_WORKERS = NUM_CORES * NUM_SUBCORES
B_PER_W = BATCH // NUM_WORKERS
CHUNK = 128
NCHUNK = B_PER_W // CHUNK
LANES = 16
BLOCKS_PER_CHUNK = CHUNK // LANES
SUPER = 2 * EMBED_DIM


def _body(tgt_idx_hbm, ctx_idx_hbm, tgt_tab_hbm, ctx_tab_hbm, out_hbm,
          tidx_v, cidx_v, tsup_v, csup_v,
          trows_a, trows_b, crows_a, crows_b, out_v, sem_t, sem_c):
    wid = lax.axis_index("s") * NUM_CORES + lax.axis_index("c")
    base = wid * B_PER_W

    for j in range(NCHUNK):
        pltpu.sync_copy(tgt_idx_hbm.at[pl.ds(base + j * CHUNK, CHUNK)],
                        tidx_v.at[j])
        pltpu.sync_copy(ctx_idx_hbm.at[pl.ds(base + j * CHUNK, CHUNK)],
                        cidx_v.at[j])
    for j in range(NCHUNK):
        for k in range(CHUNK // LANES):
            sl = pl.ds(k * LANES, LANES)
            tsup_v[j, sl] = tidx_v[j, sl] >> 1
            csup_v[j, sl] = cidx_v[j, sl] >> 1

    tbufs = (trows_a, trows_b)
    cbufs = (crows_a, crows_b)

    def fire(j):
        return (pltpu.async_copy(tgt_tab_hbm.at[tsup_v.at[j]],
                                 tbufs[j % 2], sem_t),
                pltpu.async_copy(ctx_tab_hbm.at[csup_v.at[j]],
                                 cbufs[j % 2], sem_c))

    inflight = fire(0)
    for j in range(NCHUNK):
        cur = inflight
        if j + 1 < NCHUNK:
            nxt = fire(j + 1)
        cur[0].wait()
        cur[1].wait()
        if j + 1 < NCHUNK:
            inflight = nxt
        trows = tbufs[j % 2]
        crows = cbufs[j % 2]

        def block_step(blk, _):
            sl = pl.ds(blk * LANES, LANES)
            rows16 = lax.iota(jnp.int32, LANES) + blk * LANES
            tcol0 = (tidx_v[j, sl] & 1) << 6
            ccol0 = (cidx_v[j, sl] & 1) << 6
            acc0 = jnp.zeros((LANES,), jnp.float32)
            acc1 = jnp.zeros((LANES,), jnp.float32)
            for d in range(0, EMBED_DIM, 2):
                t0 = plsc.load_gather(trows, [rows16, tcol0 + d])
                c0 = plsc.load_gather(crows, [rows16, ccol0 + d])
                t1 = plsc.load_gather(trows, [rows16, tcol0 + (d + 1)])
                c1 = plsc.load_gather(crows, [rows16, ccol0 + (d + 1)])
                acc0 = acc0 + t0 * c0
                acc1 = acc1 + t1 * c1
            out_v[pl.ds(j * CHUNK + blk * LANES, LANES)] = acc0 + acc1
            return 0

        lax.fori_loop(0, BLOCKS_PER_CHUNK, block_step, 0)

    pltpu.sync_copy(out_v, out_hbm.at[pl.ds(base, B_PER_W)])


@jax.jit
def _run(target, context, target_table, context_table):
    mesh = plsc.VectorSubcoreMesh(core_axis_name="c", subcore_axis_name="s")
    kfn = pl.kernel(
        _body,
        mesh=mesh,
        compiler_params=pltpu.CompilerParams(
            needs_layout_passes=False, use_tc_tiling_on_sc=True),
        out_type=jax.ShapeDtypeStruct((BATCH,), jnp.float32),
        scratch_types=[
            pltpu.VMEM((NCHUNK, CHUNK), jnp.int32),
            pltpu.VMEM((NCHUNK, CHUNK), jnp.int32),
            pltpu.VMEM((NCHUNK, CHUNK), jnp.int32),
            pltpu.VMEM((NCHUNK, CHUNK), jnp.int32),
            pltpu.VMEM((CHUNK, SUPER), jnp.float32),
            pltpu.VMEM((CHUNK, SUPER), jnp.float32),
            pltpu.VMEM((CHUNK, SUPER), jnp.float32),
            pltpu.VMEM((CHUNK, SUPER), jnp.float32),
            pltpu.VMEM((B_PER_W,), jnp.float32),
            pltpu.SemaphoreType.DMA,
            pltpu.SemaphoreType.DMA,
        ],
    )
    tt = target_table.reshape(VOCAB // 2, SUPER)
    ct = context_table.reshape(VOCAB // 2, SUPER)
    return kfn(target, context, tt, ct)


def kernel(target, context, target_table, context_table):
    return _run(target.astype(jnp.int32), context.astype(jnp.int32),
                target_table, context_table)


# speedup vs baseline: 1.5284x; 1.5219x over previous
import jax
import jax.numpy as jnp
from jax import lax
from jax.experimental import pallas as pl
from jax.experimental.pallas import tpu as pltpu
from jax.experimental.pallas import tpu_sc as plsc

VOCAB = 1000000
EMBED_DIM = 64
BATCH = 16384

NUM_CORES = 2
NUM_SUBCORES = 16
NUM_WORKERS = NUM_CORES * NUM_SUBCORES
B_PER_W = BATCH // NUM_WORKERS
LANES = 16
ROWCHUNK = 8
NCHUNK = B_PER_W // ROWCHUNK
IDX_PAD = B_PER_W + LANES


def _body(tgt_idx_hbm, ctx_idx_hbm, tgt_tab_hbm, ctx_tab_hbm, out_hbm,
          tidx_v, cidx_v, tbuf, cbuf, out_v, sem_t, sem_c):
    wid = lax.axis_index("s") * NUM_CORES + lax.axis_index("c")
    base = wid * B_PER_W

    pltpu.sync_copy(tgt_idx_hbm.at[pl.ds(base, B_PER_W)],
                    tidx_v.at[pl.ds(0, B_PER_W)])
    pltpu.sync_copy(ctx_idx_hbm.at[pl.ds(base, B_PER_W)],
                    cidx_v.at[pl.ds(0, B_PER_W)])
    zero16 = jnp.zeros((LANES,), jnp.int32)
    tidx_v[pl.ds(B_PER_W, LANES)] = zero16
    cidx_v[pl.ds(B_PER_W, LANES)] = zero16

    def chunk(ch, _):
        sl = pl.ds(ch * ROWCHUNK, LANES)
        tvec = tidx_v[sl]
        cvec = cidx_v[sl]
        ttile = (tvec >> 7) << 7
        ctile = (cvec >> 7) << 7
        tsub = tvec & 127 & ~15
        csub = cvec & 127 & ~15

        for k in range(ROWCHUNK):
            tt = pl.multiple_of(ttile[k], 128)
            ct = pl.multiple_of(ctile[k], 128)
            ts = tsub[k]
            cs = csub[k]
            w = pl.ds(k * 16, 16)
            for sval in range(0, 128, 16):
                @pl.when(ts == sval)
                def _(sval=sval, tt=tt, w=w):
                    pltpu.make_async_copy(
                        tgt_tab_hbm.at[:, :, pl.ds(tt, 128)]
                        .at[:, :, pl.ds(sval, 16)],
                        tbuf.at[:, :, w], sem_t).start()

                @pl.when(cs == sval)
                def _(sval=sval, ct=ct, w=w):
                    pltpu.make_async_copy(
                        ctx_tab_hbm.at[:, :, pl.ds(ct, 128)]
                        .at[:, :, pl.ds(sval, 16)],
                        cbuf.at[:, :, w], sem_c).start()

        for k in range(ROWCHUNK):
            w = pl.ds(k * 16, 16)
            pltpu.make_async_copy(
                tgt_tab_hbm.at[:, :, pl.ds(0, 128)].at[:, :, pl.ds(0, 16)],
                tbuf.at[:, :, w], sem_t).wait()
            pltpu.make_async_copy(
                ctx_tab_hbm.at[:, :, pl.ds(0, 128)].at[:, :, pl.ds(0, 16)],
                cbuf.at[:, :, w], sem_c).wait()

        ii = lax.iota(jnp.int32, LANES)
        k8 = ii & 7
        tslot = (k8 << 4) + (tvec & 15)
        cslot = (k8 << 4) + (cvec & 15)
        acc0 = jnp.zeros((LANES,), jnp.float32)
        acc1 = jnp.zeros((LANES,), jnp.float32)
        for c in range(0, EMBED_DIM, 2):
            a0 = jnp.full((LANES,), c // 8, jnp.int32)
            b0 = jnp.full((LANES,), c % 8, jnp.int32)
            a1 = jnp.full((LANES,), (c + 1) // 8, jnp.int32)
            b1 = jnp.full((LANES,), (c + 1) % 8, jnp.int32)
            t0 = plsc.load_gather(tbuf, [a0, b0, tslot])
            c0 = plsc.load_gather(cbuf, [a0, b0, cslot])
            t1 = plsc.load_gather(tbuf, [a1, b1, tslot])
            c1 = plsc.load_gather(cbuf, [a1, b1, cslot])
            acc0 = acc0 + t0 * c0
            acc1 = acc1 + t1 * c1
        plsc.store_scatter(out_v, [ch * ROWCHUNK + k8], acc0 + acc1,
                           mask=ii < ROWCHUNK)
        return 0

    lax.fori_loop(0, NCHUNK, chunk, 0)

    pltpu.sync_copy(out_v, out_hbm.at[pl.ds(base, B_PER_W)])


@jax.jit
def _run(target, context, target_table, context_table):
    mesh = plsc.VectorSubcoreMesh(core_axis_name="c", subcore_axis_name="s")
    kfn = pl.kernel(
        _body,
        mesh=mesh,
        compiler_params=pltpu.CompilerParams(
            needs_layout_passes=False, use_tc_tiling_on_sc=True),
        out_type=jax.ShapeDtypeStruct((BATCH,), jnp.float32),
        scratch_types=[
            pltpu.VMEM((IDX_PAD,), jnp.int32),
            pltpu.VMEM((IDX_PAD,), jnp.int32),
            pltpu.VMEM((8, 8, 128), jnp.float32),
            pltpu.VMEM((8, 8, 128), jnp.float32),
            pltpu.VMEM((B_PER_W,), jnp.float32),
            pltpu.SemaphoreType.DMA,
            pltpu.SemaphoreType.DMA,
        ],
    )
    tt = target_table.T.reshape(8, 8, VOCAB)
    ct = context_table.T.reshape(8, 8, VOCAB)
    return kfn(target, context, tt, ct)


def kernel(target, context, target_table, context_table):
    return _run(target.astype(jnp.int32), context.astype(jnp.int32),
                target_table, context_table)


# speedup vs baseline: 2.9600x; 1.9366x over previous
import jax
import jax.numpy as jnp
from jax import lax
from jax.experimental import pallas as pl
from jax.experimental.pallas import tpu as pltpu
from jax.experimental.pallas import tpu_sc as plsc

VOCAB = 1000000
EMBED_DIM = 64
BATCH = 16384

NUM_CORES = 2
NUM_SUBCORES = 16
NUM_WORKERS = NUM_CORES * NUM_SUBCORES
B_PER_W = BATCH // NUM_WORKERS
LANES = 16
ROWCHUNK = 4
NCHUNK = B_PER_W // ROWCHUNK
IDX_PAD = B_PER_W + LANES


def _body(tgt_idx_hbm, ctx_idx_hbm, tgt_tab_hbm, ctx_tab_hbm, out_hbm,
          tidx_v, cidx_v, tbuf0, tbuf1, cbuf0, cbuf1, out_v,
          sem_t0, sem_t1, sem_c0, sem_c1):
    wid = lax.axis_index("s") * NUM_CORES + lax.axis_index("c")
    base = wid * B_PER_W

    pltpu.sync_copy(tgt_idx_hbm.at[pl.ds(base, B_PER_W)],
                    tidx_v.at[pl.ds(0, B_PER_W)])
    pltpu.sync_copy(ctx_idx_hbm.at[pl.ds(base, B_PER_W)],
                    cidx_v.at[pl.ds(0, B_PER_W)])
    zero16 = jnp.zeros((LANES,), jnp.int32)
    tidx_v[pl.ds(B_PER_W, LANES)] = zero16
    cidx_v[pl.ds(B_PER_W, LANES)] = zero16

    tbufs = (tbuf0, tbuf1)
    cbufs = (cbuf0, cbuf1)
    tsems = (sem_t0, sem_t1)
    csems = (sem_c0, sem_c1)

    def fire(ch, slot):
        sl = pl.ds(ch * ROWCHUNK, LANES)
        tvec = tidx_v[sl]
        cvec = cidx_v[sl]
        ttile = (tvec >> 7) << 7
        ctile = (cvec >> 7) << 7
        tsub = tvec & 127 & ~31
        csub = cvec & 127 & ~31
        for k in range(ROWCHUNK):
            tt = pl.multiple_of(ttile[k], 128)
            ct = pl.multiple_of(ctile[k], 128)
            ts = tsub[k]
            cs = csub[k]
            w = pl.ds(k * 32, 32)
            for sval in range(0, 128, 32):
                @pl.when(ts == sval)
                def _(sval=sval, tt=tt, w=w, slot=slot):
                    pltpu.make_async_copy(
                        tgt_tab_hbm.at[:, :, pl.ds(tt, 128)]
                        .at[:, :, pl.ds(sval, 32)],
                        tbufs[slot].at[:, :, w], tsems[slot]).start()

                @pl.when(cs == sval)
                def _(sval=sval, ct=ct, w=w, slot=slot):
                    pltpu.make_async_copy(
                        ctx_tab_hbm.at[:, :, pl.ds(ct, 128)]
                        .at[:, :, pl.ds(sval, 32)],
                        cbufs[slot].at[:, :, w], csems[slot]).start()

    def drain(slot):
        for k in range(ROWCHUNK):
            w = pl.ds(k * 32, 32)
            pltpu.make_async_copy(
                tgt_tab_hbm.at[:, :, pl.ds(0, 128)].at[:, :, pl.ds(0, 32)],
                tbufs[slot].at[:, :, w], tsems[slot]).wait()
            pltpu.make_async_copy(
                ctx_tab_hbm.at[:, :, pl.ds(0, 128)].at[:, :, pl.ds(0, 32)],
                cbufs[slot].at[:, :, w], csems[slot]).wait()

    def compute(ch, slot):
        sl = pl.ds(ch * ROWCHUNK, LANES)
        tvec = tidx_v[sl]
        cvec = cidx_v[sl]
        ii = lax.iota(jnp.int32, LANES)
        k4 = ii & 3
        tslot = (k4 << 5) + (tvec & 31)
        cslot = (k4 << 5) + (cvec & 31)
        tb = tbufs[slot]
        cb = cbufs[slot]
        acc0 = jnp.zeros((LANES,), jnp.float32)
        acc1 = jnp.zeros((LANES,), jnp.float32)
        for c in range(0, EMBED_DIM, 2):
            a0 = jnp.full((LANES,), c // 8, jnp.int32)
            b0 = jnp.full((LANES,), c % 8, jnp.int32)
            a1 = jnp.full((LANES,), (c + 1) // 8, jnp.int32)
            b1 = jnp.full((LANES,), (c + 1) % 8, jnp.int32)
            t0 = plsc.load_gather(tb, [a0, b0, tslot])
            c0 = plsc.load_gather(cb, [a0, b0, cslot])
            t1 = plsc.load_gather(tb, [a1, b1, tslot])
            c1 = plsc.load_gather(cb, [a1, b1, cslot])
            acc0 = acc0 + t0 * c0
            acc1 = acc1 + t1 * c1
        plsc.store_scatter(out_v, [ch * ROWCHUNK + k4], acc0 + acc1,
                           mask=ii < ROWCHUNK)

    fire(0, 0)

    def two_chunks(j, _):
        ch0 = j * 2
        fire(ch0 + 1, 1)
        drain(0)
        compute(ch0, 0)

        @pl.when(j < NCHUNK // 2 - 1)
        def _():
            fire(ch0 + 2, 0)

        drain(1)
        compute(ch0 + 1, 1)
        return 0

    lax.fori_loop(0, NCHUNK // 2, two_chunks, 0)

    pltpu.sync_copy(out_v, out_hbm.at[pl.ds(base, B_PER_W)])


@jax.jit
def _run(target, context, target_table, context_table):
    mesh = plsc.VectorSubcoreMesh(core_axis_name="c", subcore_axis_name="s")
    kfn = pl.kernel(
        _body,
        mesh=mesh,
        compiler_params=pltpu.CompilerParams(
            needs_layout_passes=False, use_tc_tiling_on_sc=True),
        out_type=jax.ShapeDtypeStruct((BATCH,), jnp.float32),
        scratch_types=[
            pltpu.VMEM((IDX_PAD,), jnp.int32),
            pltpu.VMEM((IDX_PAD,), jnp.int32),
            pltpu.VMEM((8, 8, 128), jnp.float32),
            pltpu.VMEM((8, 8, 128), jnp.float32),
            pltpu.VMEM((8, 8, 128), jnp.float32),
            pltpu.VMEM((8, 8, 128), jnp.float32),
            pltpu.VMEM((B_PER_W,), jnp.float32),
            pltpu.SemaphoreType.DMA,
            pltpu.SemaphoreType.DMA,
            pltpu.SemaphoreType.DMA,
            pltpu.SemaphoreType.DMA,
        ],
    )
    tt = target_table.T.reshape(8, 8, VOCAB)
    ct = context_table.T.reshape(8, 8, VOCAB)
    return kfn(target, context, tt, ct)


def kernel(target, context, target_table, context_table):
    return _run(target.astype(jnp.int32), context.astype(jnp.int32),
                target_table, context_table)


# speedup vs baseline: 4.2210x; 1.4260x over previous
import jax
import jax.numpy as jnp
from jax import lax
from jax.experimental import pallas as pl
from jax.experimental.pallas import tpu as pltpu
from jax.experimental.pallas import tpu_sc as plsc

VOCAB = 1000000
EMBED_DIM = 64
BATCH = 16384

NUM_CORES = 2
NUM_SUBCORES = 16
NUM_WORKERS = NUM_CORES * NUM_SUBCORES
B_PER_W = BATCH // NUM_WORKERS
LANES = 16
ROWCHUNK = 4
NCHUNK = B_PER_W // ROWCHUNK
IDX_PAD = B_PER_W + LANES


def _body(tgt_idx_hbm, ctx_idx_hbm, tgt_tab_hbm, ctx_tab_hbm, out_hbm,
          tidx_v, cidx_v, tbuf0, tbuf1, cbuf0, cbuf1, out_v,
          sem_t0, sem_t1, sem_c0, sem_c1):
    wid = lax.axis_index("s") * NUM_CORES + lax.axis_index("c")
    base = wid * B_PER_W

    pltpu.sync_copy(tgt_idx_hbm.at[pl.ds(base, B_PER_W)],
                    tidx_v.at[pl.ds(0, B_PER_W)])
    pltpu.sync_copy(ctx_idx_hbm.at[pl.ds(base, B_PER_W)],
                    cidx_v.at[pl.ds(0, B_PER_W)])
    zero16 = jnp.zeros((LANES,), jnp.int32)
    tidx_v[pl.ds(B_PER_W, LANES)] = zero16
    cidx_v[pl.ds(B_PER_W, LANES)] = zero16

    tbufs = (tbuf0, tbuf1)
    cbufs = (cbuf0, cbuf1)
    tsems = (sem_t0, sem_t1)
    csems = (sem_c0, sem_c1)

    def fire(ch, slot):
        sl = pl.ds(ch * ROWCHUNK, LANES)
        tvec = tidx_v[sl]
        cvec = cidx_v[sl]
        ttile = (tvec >> 7) << 7
        ctile = (cvec >> 7) << 7
        tsub = tvec & 127 & ~63
        csub = cvec & 127 & ~63
        for k in range(ROWCHUNK):
            tt = pl.multiple_of(ttile[k], 128)
            ct = pl.multiple_of(ctile[k], 128)
            ts = tsub[k]
            cs = csub[k]
            w = pl.ds(k * 64, 64)
            for sval in range(0, 128, 64):
                @pl.when(ts == sval)
                def _(sval=sval, tt=tt, w=w, slot=slot):
                    pltpu.make_async_copy(
                        tgt_tab_hbm.at[:, :, pl.ds(tt, 128)]
                        .at[:, :, pl.ds(sval, 64)],
                        tbufs[slot].at[:, :, w], tsems[slot]).start()

                @pl.when(cs == sval)
                def _(sval=sval, ct=ct, w=w, slot=slot):
                    pltpu.make_async_copy(
                        ctx_tab_hbm.at[:, :, pl.ds(ct, 128)]
                        .at[:, :, pl.ds(sval, 64)],
                        cbufs[slot].at[:, :, w], csems[slot]).start()

    def drain(slot):
        for k in range(ROWCHUNK):
            w = pl.ds(k * 64, 64)
            pltpu.make_async_copy(
                tgt_tab_hbm.at[:, :, pl.ds(0, 128)].at[:, :, pl.ds(0, 64)],
                tbufs[slot].at[:, :, w], tsems[slot]).wait()
            pltpu.make_async_copy(
                ctx_tab_hbm.at[:, :, pl.ds(0, 128)].at[:, :, pl.ds(0, 64)],
                cbufs[slot].at[:, :, w], csems[slot]).wait()

    def compute(ch, slot):
        sl = pl.ds(ch * ROWCHUNK, LANES)
        tvec = tidx_v[sl]
        cvec = cidx_v[sl]
        ii = lax.iota(jnp.int32, LANES)
        k4 = ii & 3
        tslot = (k4 << 6) + (tvec & 63)
        cslot = (k4 << 6) + (cvec & 63)
        tb = tbufs[slot]
        cb = cbufs[slot]
        acc0 = jnp.zeros((LANES,), jnp.float32)
        acc1 = jnp.zeros((LANES,), jnp.float32)
        for c in range(0, EMBED_DIM, 2):
            a0 = jnp.full((LANES,), c // 8, jnp.int32)
            b0 = jnp.full((LANES,), c % 8, jnp.int32)
            a1 = jnp.full((LANES,), (c + 1) // 8, jnp.int32)
            b1 = jnp.full((LANES,), (c + 1) % 8, jnp.int32)
            t0 = plsc.load_gather(tb, [a0, b0, tslot])
            c0 = plsc.load_gather(cb, [a0, b0, cslot])
            t1 = plsc.load_gather(tb, [a1, b1, tslot])
            c1 = plsc.load_gather(cb, [a1, b1, cslot])
            acc0 = acc0 + t0 * c0
            acc1 = acc1 + t1 * c1
        plsc.store_scatter(out_v, [ch * ROWCHUNK + k4], acc0 + acc1,
                           mask=ii < ROWCHUNK)

    fire(0, 0)

    def two_chunks(j, _):
        ch0 = j * 2
        fire(ch0 + 1, 1)
        drain(0)
        compute(ch0, 0)

        @pl.when(j < NCHUNK // 2 - 1)
        def _():
            fire(ch0 + 2, 0)

        drain(1)
        compute(ch0 + 1, 1)
        return 0

    lax.fori_loop(0, NCHUNK // 2, two_chunks, 0)

    pltpu.sync_copy(out_v, out_hbm.at[pl.ds(base, B_PER_W)])


@jax.jit
def _run(target, context, target_table, context_table):
    mesh = plsc.VectorSubcoreMesh(core_axis_name="c", subcore_axis_name="s")
    kfn = pl.kernel(
        _body,
        mesh=mesh,
        compiler_params=pltpu.CompilerParams(
            needs_layout_passes=False, use_tc_tiling_on_sc=True),
        out_type=jax.ShapeDtypeStruct((BATCH,), jnp.float32),
        scratch_types=[
            pltpu.VMEM((IDX_PAD,), jnp.int32),
            pltpu.VMEM((IDX_PAD,), jnp.int32),
            pltpu.VMEM((8, 8, 256), jnp.float32),
            pltpu.VMEM((8, 8, 256), jnp.float32),
            pltpu.VMEM((8, 8, 256), jnp.float32),
            pltpu.VMEM((8, 8, 256), jnp.float32),
            pltpu.VMEM((B_PER_W,), jnp.float32),
            pltpu.SemaphoreType.DMA,
            pltpu.SemaphoreType.DMA,
            pltpu.SemaphoreType.DMA,
            pltpu.SemaphoreType.DMA,
        ],
    )
    tt = target_table.T.reshape(8, 8, VOCAB)
    ct = context_table.T.reshape(8, 8, VOCAB)
    return kfn(target, context, tt, ct)


def kernel(target, context, target_table, context_table):
    return _run(target.astype(jnp.int32), context.astype(jnp.int32),
                target_table, context_table)


# speedup vs baseline: 4.8948x; 1.1596x over previous
import jax
import jax.numpy as jnp
from jax import lax
from jax.experimental import pallas as pl
from jax.experimental.pallas import tpu as pltpu
from jax.experimental.pallas import tpu_sc as plsc

VOCAB = 1000000
EMBED_DIM = 64
BATCH = 16384

NUM_CORES = 2
NUM_SUBCORES = 16
NUM_WORKERS = NUM_CORES * NUM_SUBCORES
B_PER_W = BATCH // NUM_WORKERS
LANES = 16
ROWCHUNK = 4
NCHUNK = B_PER_W // ROWCHUNK
IDX_PAD = B_PER_W + LANES


def _body(tgt_idx_hbm, ctx_idx_hbm, tgt_tab_hbm, ctx_tab_hbm, out_hbm,
          tidx_v, cidx_v, tbuf0, tbuf1, cbuf0, cbuf1, out_v,
          sem_t0, sem_t1, sem_c0, sem_c1):
    wid = lax.axis_index("s") * NUM_CORES + lax.axis_index("c")
    base = wid * B_PER_W

    pltpu.sync_copy(tgt_idx_hbm.at[pl.ds(base, B_PER_W)],
                    tidx_v.at[pl.ds(0, B_PER_W)])
    pltpu.sync_copy(ctx_idx_hbm.at[pl.ds(base, B_PER_W)],
                    cidx_v.at[pl.ds(0, B_PER_W)])
    zero16 = jnp.zeros((LANES,), jnp.int32)
    tidx_v[pl.ds(B_PER_W, LANES)] = zero16
    cidx_v[pl.ds(B_PER_W, LANES)] = zero16

    tbufs = (tbuf0, tbuf1)
    cbufs = (cbuf0, cbuf1)
    tsems = (sem_t0, sem_t1)
    csems = (sem_c0, sem_c1)

    def fire(ch, slot):
        sl = pl.ds(ch * ROWCHUNK, LANES)
        tvec = tidx_v[sl]
        cvec = cidx_v[sl]
        ttile = (tvec >> 7) << 7
        ctile = (cvec >> 7) << 7
        tsub = tvec & 127 & ~63
        csub = cvec & 127 & ~63
        for k in range(ROWCHUNK):
            tt = pl.multiple_of(ttile[k], 128)
            ct = pl.multiple_of(ctile[k], 128)
            ts = tsub[k]
            cs = csub[k]
            w = pl.ds(k * 64, 64)
            for sval in range(0, 128, 64):
                @pl.when(ts == sval)
                def _(sval=sval, tt=tt, w=w, slot=slot):
                    pltpu.make_async_copy(
                        tgt_tab_hbm.at[:, :, pl.ds(tt, 128)]
                        .at[:, :, pl.ds(sval, 64)],
                        tbufs[slot].at[:, :, w], tsems[slot]).start()

                @pl.when(cs == sval)
                def _(sval=sval, ct=ct, w=w, slot=slot):
                    pltpu.make_async_copy(
                        ctx_tab_hbm.at[:, :, pl.ds(ct, 128)]
                        .at[:, :, pl.ds(sval, 64)],
                        cbufs[slot].at[:, :, w], csems[slot]).start()

    def drain(slot):
        pltpu.make_async_copy(
            tgt_tab_hbm.at[:, :, pl.ds(0, 256)],
            tbufs[slot], tsems[slot]).wait()
        pltpu.make_async_copy(
            ctx_tab_hbm.at[:, :, pl.ds(0, 256)],
            cbufs[slot], csems[slot]).wait()

    def compute(ch, slot):
        sl = pl.ds(ch * ROWCHUNK, LANES)
        tvec = tidx_v[sl]
        cvec = cidx_v[sl]
        ii = lax.iota(jnp.int32, LANES)
        row = ii & 3
        feat = ii >> 2
        twin = (tvec & 63).at[row].get(mode="promise_in_bounds")
        cwin = (cvec & 63).at[row].get(mode="promise_in_bounds")
        tslot = (row << 6) + twin
        cslot = (row << 6) + cwin
        tb = tbufs[slot]
        cb = cbufs[slot]
        acc = jnp.zeros((LANES,), jnp.float32)
        for c0 in range(0, EMBED_DIM, 4):
            a0 = jnp.full((LANES,), c0 // 8, jnp.int32)
            b0 = (c0 % 8) + feat
            tv = plsc.load_gather(tb, [a0, b0, tslot])
            cv = plsc.load_gather(cb, [a0, b0, cslot])
            acc = acc + tv * cv
        acc = acc + acc.at[ii ^ 8].get(mode="promise_in_bounds")
        acc = acc + acc.at[ii ^ 4].get(mode="promise_in_bounds")
        plsc.store_scatter(out_v, [ch * ROWCHUNK + row], acc,
                           mask=ii < ROWCHUNK)

    fire(0, 0)

    def two_chunks(j, _):
        ch0 = j * 2
        fire(ch0 + 1, 1)
        drain(0)
        compute(ch0, 0)

        @pl.when(j < NCHUNK // 2 - 1)
        def _():
            fire(ch0 + 2, 0)

        drain(1)
        compute(ch0 + 1, 1)
        return 0

    lax.fori_loop(0, NCHUNK // 2, two_chunks, 0)

    pltpu.sync_copy(out_v, out_hbm.at[pl.ds(base, B_PER_W)])


@jax.jit
def _run(target, context, target_table, context_table):
    mesh = plsc.VectorSubcoreMesh(core_axis_name="c", subcore_axis_name="s")
    kfn = pl.kernel(
        _body,
        mesh=mesh,
        compiler_params=pltpu.CompilerParams(
            needs_layout_passes=False, use_tc_tiling_on_sc=True),
        out_type=jax.ShapeDtypeStruct((BATCH,), jnp.float32),
        scratch_types=[
            pltpu.VMEM((IDX_PAD,), jnp.int32),
            pltpu.VMEM((IDX_PAD,), jnp.int32),
            pltpu.VMEM((8, 8, 256), jnp.float32),
            pltpu.VMEM((8, 8, 256), jnp.float32),
            pltpu.VMEM((8, 8, 256), jnp.float32),
            pltpu.VMEM((8, 8, 256), jnp.float32),
            pltpu.VMEM((B_PER_W,), jnp.float32),
            pltpu.SemaphoreType.DMA,
            pltpu.SemaphoreType.DMA,
            pltpu.SemaphoreType.DMA,
            pltpu.SemaphoreType.DMA,
        ],
    )
    tt = target_table.T.reshape(8, 8, VOCAB)
    ct = context_table.T.reshape(8, 8, VOCAB)
    return kfn(target, context, tt, ct)


def kernel(target, context, target_table, context_table):
    return _run(target.astype(jnp.int32), context.astype(jnp.int32),
                target_table, context_table)
